# baseline (speedup 1.0000x reference)
import jax
import jax.numpy as jnp
from jax import lax
from jax.experimental import pallas as pl
from jax.experimental.pallas import tpu as pltpu

V = 1000000
D = 64
L_CTX = 200
KG = 8
NG = L_CTX // KG
CBLK = 65536
NBLK = (V + CBLK - 1) // CBLK


def _gather_body(ctx_ref, *refs):
    embt_refs = refs[:KG]
    e_ref = refs[KG]
    i = pl.program_id(0)

    @pl.when(i == 0)
    def _():
        e_ref[...] = jnp.zeros_like(e_ref)

    col = lax.broadcasted_iota(jnp.int32, (D, 128), 1)
    acc = e_ref[...]
    for j in range(KG):
        lane = ctx_ref[KG * i + j] % 128
        acc = acc + jnp.where(col == lane, embt_refs[j][...], 0.0)
    e_ref[...] = acc

    @pl.when(i == NG - 1)
    def _():
        tot = jnp.sum(e_ref[...], axis=1, keepdims=True) * (1.0 / L_CTX)
        e_ref[...] = jnp.broadcast_to(tot, (D, 128))


def _matvec_body(e_ref, wt_ref, wt2_ref, b_ref, o_ref, m_ref, s_ref, zs_ref):
    g = pl.program_id(0)

    @pl.when(g == 0)
    def _():
        m_ref[0] = -jnp.inf
        s_ref[0] = 0.0

    @pl.when(g < NBLK)
    def _phase1():
        z0 = lax.dot_general(
            e_ref[0:32, :], wt_ref[...], (((0,), (0,)), ((), ())),
            preferred_element_type=jnp.float32,
        )
        z1 = lax.dot_general(
            e_ref[32:64, :], wt2_ref[...], (((0,), (0,)), ((), ())),
            preferred_element_type=jnp.float32,
        )
        z = z0 + z1 + b_ref[...][None, :]
        col = g * CBLK + lax.broadcasted_iota(jnp.int32, (1, CBLK), 1)
        z = jnp.where(col < V, z, -jnp.inf)
        zs_ref[pl.ds(g * CBLK, CBLK)] = z[0]
        m_old = m_ref[0]
        m_new = jnp.maximum(m_old, jnp.max(z))
        bsum = jnp.sum(jnp.exp(z - m_new))
        s_ref[0] = s_ref[0] * jnp.exp(m_old - m_new) + bsum
        m_ref[0] = m_new

    @pl.when(g >= NBLK)
    def _phase2():
        lse = m_ref[0] + jnp.log(s_ref[0])
        o_ref[...] = zs_ref[pl.ds((g - NBLK) * CBLK, CBLK)] - lse


def kernel(context, emb, W, b):
    ctx = context.astype(jnp.int32)
    embt = emb.T
    wt = W.T

    def _tile_spec(j):
        return pl.BlockSpec((D, 128), lambda i, c, j=j: (0, c[KG * i + j] // 128))

    e_wide = pl.pallas_call(
        _gather_body,
        grid_spec=pltpu.PrefetchScalarGridSpec(
            num_scalar_prefetch=1,
            grid=(NG,),
            in_specs=[_tile_spec(j) for j in range(KG)],
            out_specs=pl.BlockSpec((D, 128), lambda i, c: (0, 0)),
        ),
        out_shape=jax.ShapeDtypeStruct((D, 128), jnp.float32),
    )(ctx, *([embt] * KG))
    e2 = e_wide[:, :1]

    out = pl.pallas_call(
        _matvec_body,
        grid=(2 * NBLK,),
        in_specs=[
            pl.BlockSpec((D, 1), lambda g: (0, 0)),
            pl.BlockSpec((32, CBLK), lambda g: (0, jnp.minimum(g, NBLK - 1))),
            pl.BlockSpec((32, CBLK), lambda g: (1, jnp.minimum(g, NBLK - 1))),
            pl.BlockSpec((CBLK,), lambda g: (jnp.minimum(g, NBLK - 1),)),
        ],
        out_specs=pl.BlockSpec((CBLK,),
                               lambda g: (jnp.maximum(g - NBLK, 0),)),
        out_shape=jax.ShapeDtypeStruct((V,), jnp.float32),
        scratch_shapes=[
            pltpu.SMEM((1,), jnp.float32),
            pltpu.SMEM((1,), jnp.float32),
            pltpu.VMEM((NBLK * CBLK,), jnp.float32),
        ],
    )(e2, wt, wt, b)
    return out


# speedup vs baseline: 1.0074x; 1.0074x over previous
import jax
import jax.numpy as jnp
from jax import lax
from jax.experimental import pallas as pl
from jax.experimental.pallas import tpu as pltpu

V = 1000000
D = 64
L_CTX = 200
KG = 8
NG = L_CTX // KG
CBLK = 65536
NBLK = (V + CBLK - 1) // CBLK


def _gather_body(ctx_ref, *refs):
    embt_refs = refs[:KG]
    e_ref = refs[KG]
    i = pl.program_id(0)

    @pl.when(i == 0)
    def _():
        e_ref[...] = jnp.zeros_like(e_ref)

    col = lax.broadcasted_iota(jnp.int32, (D, 128), 1)
    acc = e_ref[...]
    for j in range(KG):
        lane = ctx_ref[KG * i + j] % 128
        acc = acc + jnp.where(col == lane, embt_refs[j][...], 0.0)
    e_ref[...] = acc

    @pl.when(i == NG - 1)
    def _():
        tot = jnp.sum(e_ref[...], axis=1, keepdims=True) * (1.0 / L_CTX)
        e_ref[...] = jnp.broadcast_to(tot, (D, 128))


def _matvec_body(e_ref, wt_ref, b_ref, o_ref, m_ref, s_ref, zs_ref):
    g = pl.program_id(0)

    @pl.when(g == 0)
    def _():
        m_ref[0] = -jnp.inf
        s_ref[0] = 0.0

    @pl.when(g < NBLK)
    def _phase1():
        z = lax.dot_general(
            e_ref[...], wt_ref[...], (((0,), (0,)), ((), ())),
            preferred_element_type=jnp.float32,
        )
        z = z + b_ref[...][None, :]
        col = g * CBLK + lax.broadcasted_iota(jnp.int32, (1, CBLK), 1)
        z = jnp.where(col < V, z, -jnp.inf)
        zs_ref[pl.ds(g * CBLK, CBLK)] = z[0]
        m_old = m_ref[0]
        m_new = jnp.maximum(m_old, jnp.max(z))
        bsum = jnp.sum(jnp.exp(z - m_new))
        s_ref[0] = s_ref[0] * jnp.exp(m_old - m_new) + bsum
        m_ref[0] = m_new

    @pl.when(g >= NBLK)
    def _phase2():
        lse = m_ref[0] + jnp.log(s_ref[0])
        o_ref[...] = zs_ref[pl.ds((g - NBLK) * CBLK, CBLK)] - lse


def kernel(context, emb, W, b):
    ctx = context.astype(jnp.int32)
    embt = emb.T
    wt = W.T

    def _tile_spec(j):
        return pl.BlockSpec((D, 128), lambda i, c, j=j: (0, c[KG * i + j] // 128))

    e_wide = pl.pallas_call(
        _gather_body,
        grid_spec=pltpu.PrefetchScalarGridSpec(
            num_scalar_prefetch=1,
            grid=(NG,),
            in_specs=[_tile_spec(j) for j in range(KG)],
            out_specs=pl.BlockSpec((D, 128), lambda i, c: (0, 0)),
        ),
        out_shape=jax.ShapeDtypeStruct((D, 128), jnp.float32),
    )(ctx, *([embt] * KG))
    e2 = e_wide[:, :1]

    out = pl.pallas_call(
        _matvec_body,
        grid=(2 * NBLK,),
        in_specs=[
            pl.BlockSpec((D, 1), lambda g: (0, 0)),
            pl.BlockSpec((D, CBLK), lambda g: (0, jnp.minimum(g, NBLK - 1))),
            pl.BlockSpec((CBLK,), lambda g: (jnp.minimum(g, NBLK - 1),)),
        ],
        out_specs=pl.BlockSpec((CBLK,),
                               lambda g: (jnp.maximum(g - NBLK, 0),)),
        out_shape=jax.ShapeDtypeStruct((V,), jnp.float32),
        scratch_shapes=[
            pltpu.SMEM((1,), jnp.float32),
            pltpu.SMEM((1,), jnp.float32),
            pltpu.VMEM((NBLK * CBLK,), jnp.float32),
        ],
    )(e2, wt, b)
    return out
